# Initial kernel scaffold; baseline (speedup 1.0000x reference)
import jax
import jax.numpy as jnp
from jax.experimental import pallas as pl


def kernel(X, hg, v2e_weight, e2v_weight, W1, b1, W2, b2):
    raise NotImplementedError("write your pallas kernel here")


# speedup vs baseline: 3.2227x; 3.2227x over previous
import functools

import jax
import jax.numpy as jnp
from jax import lax
from jax.experimental import pallas as pl
from jax.experimental.pallas import tpu as pltpu
from jax.experimental.pallas import tpu_sc as plsc

NV = 10000
NP = 10240
NNZ = 320000
CHUNK = 128
NCHUNK = 80
PER_TILE = CHUNK * NCHUNK
NNZP = PER_TILE * 32
RPT = NP // 16
BLK = 256


def _sc_mesh():
    return plsc.VectorSubcoreMesh(core_axis_name="c", subcore_axis_name="s")


@functools.cache
def _degree_kernel():
    out = [jax.ShapeDtypeStruct((NP, 16), jnp.float32) for _ in range(4)]
    scratch = [
        pltpu.VMEM((NCHUNK, CHUNK), jnp.int32),
        pltpu.VMEM((NCHUNK, CHUNK), jnp.int32),
        pltpu.VMEM((NCHUNK, CHUNK), jnp.float32),
        pltpu.VMEM((NCHUNK, CHUNK), jnp.float32),
        pltpu.VMEM((CHUNK, 16), jnp.float32),
        pltpu.VMEM((CHUNK, 16), jnp.float32),
        pltpu.VMEM_SHARED((NP, 16), jnp.float32),
        pltpu.VMEM_SHARED((NP, 16), jnp.float32),
    ]

    @functools.partial(
        pl.kernel, out_type=out, mesh=_sc_mesh(), scratch_types=scratch,
        compiler_params=pltpu.CompilerParams(use_tc_tiling_on_sc=False))
    def k(vid_h, eid_h, wv_h, we_h, de0, de1, dv0, dv1,
          eidb, vidb, wvb, web, mbe, mbv, acc_de, acc_dv):
        cid = lax.axis_index("c")
        sid = lax.axis_index("s")
        pltpu.sync_copy(eid_h.at[cid, sid], eidb)
        pltpu.sync_copy(vid_h.at[cid, sid], vidb)
        pltpu.sync_copy(wv_h.at[cid, sid], wvb)
        pltpu.sync_copy(we_h.at[cid, sid], web)

        zero = jnp.zeros((16,), jnp.float32)

        @pl.loop(0, CHUNK)
        def _(r):
            mbe[r, :] = zero
        base = sid * RPT
        for kk in range(RPT // CHUNK):
            pltpu.sync_copy(mbe, acc_de.at[pl.ds(base + kk * CHUNK, CHUNK)])
            pltpu.sync_copy(mbe, acc_dv.at[pl.ds(base + kk * CHUNK, CHUNK)])
        plsc.subcore_barrier()

        @pl.loop(0, NCHUNK)
        def _(g):
            @pl.loop(0, CHUNK // 16)
            def _(kk):
                wv_vec = wvb[g, pl.ds(kk * 16, 16)]
                we_vec = web[g, pl.ds(kk * 16, 16)]
                for i in range(16):
                    mbe[kk * 16 + i, :] = jnp.full((16,), wv_vec[i],
                                                   jnp.float32)
                    mbv[kk * 16 + i, :] = jnp.full((16,), we_vec[i],
                                                   jnp.float32)
            pltpu.sync_copy(mbe, acc_de.at[eidb.at[g]], add=True)
            pltpu.sync_copy(mbv, acc_dv.at[vidb.at[g]], add=True)

        plsc.subcore_barrier()
        for kk in range(RPT // CHUNK):
            sl = pl.ds(base + kk * CHUNK, CHUNK)
            pltpu.sync_copy(acc_de.at[sl], mbe)
            pltpu.sync_copy(acc_dv.at[sl], mbv)

            @pl.when(cid == 0)
            def _():
                pltpu.sync_copy(mbe, de0.at[sl])
                pltpu.sync_copy(mbv, dv0.at[sl])

            @pl.when(cid == 1)
            def _():
                pltpu.sync_copy(mbe, de1.at[sl])
                pltpu.sync_copy(mbv, dv1.at[sl])

    return k


@functools.cache
def _accum_kernel(C):
    out = [jax.ShapeDtypeStruct((NP, C), jnp.float32) for _ in range(2)]
    scratch = [
        pltpu.VMEM((NCHUNK, CHUNK), jnp.int32),
        pltpu.VMEM((NCHUNK, CHUNK), jnp.int32),
        pltpu.VMEM((NCHUNK, CHUNK), jnp.float32),
        pltpu.VMEM((CHUNK, C), jnp.float32),
        pltpu.VMEM_SHARED((NP, C), jnp.float32),
        pltpu.SemaphoreType.DMA,
    ]

    @functools.partial(
        pl.kernel, out_type=out, mesh=_sc_mesh(), scratch_types=scratch,
        compiler_params=pltpu.CompilerParams(use_tc_tiling_on_sc=False))
    def k(table_h, src_h, dst_h, w_h, out0, out1,
          srcb, dstb, wb, gbuf, acc, sem):
        cid = lax.axis_index("c")
        sid = lax.axis_index("s")
        pltpu.sync_copy(src_h.at[cid, sid], srcb)
        pltpu.sync_copy(dst_h.at[cid, sid], dstb)
        pltpu.sync_copy(w_h.at[cid, sid], wb)

        zero = jnp.zeros((16,), jnp.float32)

        @pl.loop(0, CHUNK)
        def _(r):
            for j in range(C // 16):
                gbuf[r, pl.ds(16 * j, 16)] = zero
        base = sid * RPT
        for kk in range(RPT // CHUNK):
            pltpu.sync_copy(gbuf, acc.at[pl.ds(base + kk * CHUNK, CHUNK)])
        plsc.subcore_barrier()

        @pl.loop(0, NCHUNK)
        def _(g):
            pltpu.async_copy(table_h.at[srcb.at[g]], gbuf, sem).wait()

            @pl.loop(0, CHUNK // 16)
            def _(kk):
                w_vec = wb[g, pl.ds(kk * 16, 16)]
                for i in range(16):
                    s = jnp.full((16,), w_vec[i], jnp.float32)
                    r = kk * 16 + i
                    for j in range(C // 16):
                        gbuf[r, pl.ds(16 * j, 16)] = (
                            gbuf[r, pl.ds(16 * j, 16)] * s)

            pltpu.sync_copy(gbuf, acc.at[dstb.at[g]], add=True)

        plsc.subcore_barrier()
        for kk in range(RPT // CHUNK):
            sl = pl.ds(base + kk * CHUNK, CHUNK)
            pltpu.sync_copy(acc.at[sl], gbuf)

            @pl.when(cid == 0)
            def _():
                pltpu.sync_copy(gbuf, out0.at[sl])

            @pl.when(cid == 1)
            def _():
                pltpu.sync_copy(gbuf, out1.at[sl])

    return k


def _tc_mm(x, w, b):
    n, kdim = x.shape
    c = w.shape[1]

    def body(x_ref, w_ref, b_ref, o_ref):
        o_ref[...] = (jnp.dot(x_ref[...], w_ref[...],
                              preferred_element_type=jnp.float32) + b_ref[...])

    return pl.pallas_call(
        body,
        grid=(n // BLK,),
        in_specs=[pl.BlockSpec((BLK, kdim), lambda i: (i, 0)),
                  pl.BlockSpec((kdim, c), lambda i: (0, 0)),
                  pl.BlockSpec((1, c), lambda i: (0, 0))],
        out_specs=pl.BlockSpec((BLK, c), lambda i: (i, 0)),
        out_shape=jax.ShapeDtypeStruct((n, c), jnp.float32),
    )(x, w, b)


def _tc_comb(p0, p1, d0, d1):
    n, c = p0.shape

    def body(p0r, p1r, d0r, d1r, o_ref):
        d = d0r[...] + d1r[...]
        num = p0r[...] + p1r[...]
        pos = d > 0
        o_ref[...] = jnp.where(pos, num / jnp.where(pos, d, 1.0), 0.0)

    return pl.pallas_call(
        body,
        grid=(n // BLK,),
        in_specs=[pl.BlockSpec((BLK, c), lambda i: (i, 0)),
                  pl.BlockSpec((BLK, c), lambda i: (i, 0)),
                  pl.BlockSpec((BLK, 1), lambda i: (i, 0)),
                  pl.BlockSpec((BLK, 1), lambda i: (i, 0))],
        out_specs=pl.BlockSpec((BLK, c), lambda i: (i, 0)),
        out_shape=jax.ShapeDtypeStruct((n, c), jnp.float32),
    )(p0, p1, d0, d1)


def _tc_comb_relu_mm(q0, q1, d0, d1, w, b):
    n, c = q0.shape
    c2 = w.shape[1]

    def body(q0r, q1r, d0r, d1r, wr, br, xo, to):
        d = d0r[...] + d1r[...]
        num = q0r[...] + q1r[...]
        pos = d > 0
        xv = jnp.where(pos, num / jnp.where(pos, d, 1.0), 0.0)
        xv = jnp.maximum(xv, 0.0)
        xo[...] = xv
        to[...] = (jnp.dot(xv, wr[...],
                           preferred_element_type=jnp.float32) + br[...])

    return pl.pallas_call(
        body,
        grid=(n // BLK,),
        in_specs=[pl.BlockSpec((BLK, c), lambda i: (i, 0)),
                  pl.BlockSpec((BLK, c), lambda i: (i, 0)),
                  pl.BlockSpec((BLK, 1), lambda i: (i, 0)),
                  pl.BlockSpec((BLK, 1), lambda i: (i, 0)),
                  pl.BlockSpec((c, c2), lambda i: (0, 0)),
                  pl.BlockSpec((1, c2), lambda i: (0, 0))],
        out_specs=[pl.BlockSpec((BLK, c), lambda i: (i, 0)),
                   pl.BlockSpec((BLK, c2), lambda i: (i, 0))],
        out_shape=[jax.ShapeDtypeStruct((n, c), jnp.float32),
                   jax.ShapeDtypeStruct((n, c2), jnp.float32)],
    )(q0, q1, d0, d1, w, b)


def kernel(X, hg, v2e_weight, e2v_weight, W1, b1, W2, b2):
    vid = hg[0]
    eid = hg[1]
    padn = NNZP - NNZ
    pad_idx = jnp.full((padn,), NP - 1, jnp.int32)
    pad_w = jnp.zeros((padn,), jnp.float32)
    shape4 = (2, 16, NCHUNK, CHUNK)
    vid_p = jnp.concatenate([vid, pad_idx]).reshape(shape4)
    eid_p = jnp.concatenate([eid, pad_idx]).reshape(shape4)
    wv_p = jnp.concatenate([v2e_weight, pad_w]).reshape(shape4)
    we_p = jnp.concatenate([e2v_weight, pad_w]).reshape(shape4)
    X_p = jnp.pad(X, ((0, NP - NV), (0, 0)))
    b1r = b1.reshape(1, -1)
    b2r = b2.reshape(1, -1)

    de0, de1, dv0, dv1 = _degree_kernel()(vid_p, eid_p, wv_p, we_p)
    de0c, de1c = de0[:, :1], de1[:, :1]
    dv0c, dv1c = dv0[:, :1], dv1[:, :1]

    t1 = _tc_mm(X_p, W1, b1r)
    p0, p1 = _accum_kernel(64)(t1, vid_p, eid_p, wv_p)
    xe1 = _tc_comb(p0, p1, de0c, de1c)
    q0, q1 = _accum_kernel(64)(xe1, eid_p, vid_p, we_p)
    xn1, t2 = _tc_comb_relu_mm(q0, q1, dv0c, dv1c, W2, b2r)
    r0, r1 = _accum_kernel(128)(t2, vid_p, eid_p, wv_p)
    xe2 = _tc_comb(r0, r1, de0c, de1c)
    s0, s1 = _accum_kernel(128)(xe2, eid_p, vid_p, we_p)
    xn2 = _tc_comb(s0, s1, dv0c, dv1c)

    return (xn1[:NV], xe1[:NV], xn2[:NV], xe2[:NV])


# speedup vs baseline: 3.4143x; 1.0594x over previous
import functools

import jax
import jax.numpy as jnp
from jax import lax
from jax.experimental import pallas as pl
from jax.experimental.pallas import tpu as pltpu
from jax.experimental.pallas import tpu_sc as plsc

NV = 10000
NP = 10240
NNZ = 320000
CHUNK = 128
NCHUNK = 80
PER_TILE = CHUNK * NCHUNK
NNZP = PER_TILE * 32
RPT = NP // 16
BLK = 256


def _sc_mesh():
    return plsc.VectorSubcoreMesh(core_axis_name="c", subcore_axis_name="s")


@functools.cache
def _degree_kernel():
    out = [jax.ShapeDtypeStruct((NP, 16), jnp.float32) for _ in range(4)]
    scratch = [
        pltpu.VMEM((NCHUNK, CHUNK), jnp.int32),
        pltpu.VMEM((NCHUNK, CHUNK), jnp.int32),
        pltpu.VMEM((NCHUNK, CHUNK), jnp.float32),
        pltpu.VMEM((NCHUNK, CHUNK), jnp.float32),
        pltpu.VMEM((CHUNK, 16), jnp.float32),
        pltpu.VMEM((CHUNK, 16), jnp.float32),
        pltpu.VMEM_SHARED((NP, 16), jnp.float32),
        pltpu.VMEM_SHARED((NP, 16), jnp.float32),
    ]

    @functools.partial(
        pl.kernel, out_type=out, mesh=_sc_mesh(), scratch_types=scratch,
        compiler_params=pltpu.CompilerParams(use_tc_tiling_on_sc=False))
    def k(vid_h, eid_h, wv_h, we_h, de0, de1, dv0, dv1,
          eidb, vidb, wvb, web, mbe, mbv, acc_de, acc_dv):
        cid = lax.axis_index("c")
        sid = lax.axis_index("s")
        pltpu.sync_copy(eid_h.at[cid, sid], eidb)
        pltpu.sync_copy(vid_h.at[cid, sid], vidb)
        pltpu.sync_copy(wv_h.at[cid, sid], wvb)
        pltpu.sync_copy(we_h.at[cid, sid], web)

        zero = jnp.zeros((16,), jnp.float32)

        @pl.loop(0, CHUNK)
        def _(r):
            mbe[r, :] = zero
        base = sid * RPT
        for kk in range(RPT // CHUNK):
            pltpu.sync_copy(mbe, acc_de.at[pl.ds(base + kk * CHUNK, CHUNK)])
            pltpu.sync_copy(mbe, acc_dv.at[pl.ds(base + kk * CHUNK, CHUNK)])
        plsc.subcore_barrier()

        @pl.loop(0, NCHUNK)
        def _(g):
            @pl.loop(0, CHUNK // 16)
            def _(kk):
                wv_vec = wvb[g, pl.ds(kk * 16, 16)]
                we_vec = web[g, pl.ds(kk * 16, 16)]
                for i in range(16):
                    mbe[kk * 16 + i, :] = jnp.full((16,), wv_vec[i],
                                                   jnp.float32)
                    mbv[kk * 16 + i, :] = jnp.full((16,), we_vec[i],
                                                   jnp.float32)
            pltpu.sync_copy(mbe, acc_de.at[eidb.at[g]], add=True)
            pltpu.sync_copy(mbv, acc_dv.at[vidb.at[g]], add=True)

        plsc.subcore_barrier()
        for kk in range(RPT // CHUNK):
            sl = pl.ds(base + kk * CHUNK, CHUNK)
            pltpu.sync_copy(acc_de.at[sl], mbe)
            pltpu.sync_copy(acc_dv.at[sl], mbv)

            @pl.when(cid == 0)
            def _():
                pltpu.sync_copy(mbe, de0.at[sl])
                pltpu.sync_copy(mbv, dv0.at[sl])

            @pl.when(cid == 1)
            def _():
                pltpu.sync_copy(mbe, de1.at[sl])
                pltpu.sync_copy(mbv, dv1.at[sl])

    return k


@functools.cache
def _accum_kernel(C):
    out = [jax.ShapeDtypeStruct((NP, C), jnp.float32) for _ in range(2)]
    scratch = [
        pltpu.VMEM((NCHUNK, CHUNK), jnp.int32),
        pltpu.VMEM((NCHUNK, CHUNK), jnp.int32),
        pltpu.VMEM((NCHUNK, CHUNK), jnp.float32),
        pltpu.VMEM((CHUNK, C), jnp.float32),
        pltpu.VMEM_SHARED((NP, C), jnp.float32),
        pltpu.SemaphoreType.DMA,
    ]

    @functools.partial(
        pl.kernel, out_type=out, mesh=_sc_mesh(), scratch_types=scratch,
        compiler_params=pltpu.CompilerParams(use_tc_tiling_on_sc=False))
    def k(table_h, src_h, dst_h, w_h, out0, out1,
          srcb, dstb, wb, gbuf, acc, sem):
        cid = lax.axis_index("c")
        sid = lax.axis_index("s")
        pltpu.sync_copy(src_h.at[cid, sid], srcb)
        pltpu.sync_copy(dst_h.at[cid, sid], dstb)
        pltpu.sync_copy(w_h.at[cid, sid], wb)

        zero = jnp.zeros((16,), jnp.float32)

        @pl.loop(0, CHUNK)
        def _(r):
            for j in range(C // 16):
                gbuf[r, pl.ds(16 * j, 16)] = zero
        base = sid * RPT
        for kk in range(RPT // CHUNK):
            pltpu.sync_copy(gbuf, acc.at[pl.ds(base + kk * CHUNK, CHUNK)])
        plsc.subcore_barrier()

        @pl.loop(0, NCHUNK)
        def _(g):
            pltpu.async_copy(table_h.at[srcb.at[g]], gbuf, sem).wait()

            @pl.loop(0, CHUNK // 16)
            def _(kk):
                w_vec = wb[g, pl.ds(kk * 16, 16)]
                for i in range(16):
                    s = jnp.full((16,), w_vec[i], jnp.float32)
                    r = kk * 16 + i
                    for j in range(C // 16):
                        gbuf[r, pl.ds(16 * j, 16)] = (
                            gbuf[r, pl.ds(16 * j, 16)] * s)

            pltpu.sync_copy(gbuf, acc.at[dstb.at[g]], add=True)

        plsc.subcore_barrier()
        for kk in range(RPT // CHUNK):
            sl = pl.ds(base + kk * CHUNK, CHUNK)
            pltpu.sync_copy(acc.at[sl], gbuf)

            @pl.when(cid == 0)
            def _():
                pltpu.sync_copy(gbuf, out0.at[sl])

            @pl.when(cid == 1)
            def _():
                pltpu.sync_copy(gbuf, out1.at[sl])

    return k


def _tc_mm(x, w, b):
    n, kdim = x.shape
    c = w.shape[1]

    def body(x_ref, w_ref, b_ref, o_ref):
        o_ref[...] = (jnp.dot(x_ref[...], w_ref[...],
                              preferred_element_type=jnp.float32) + b_ref[...])

    return pl.pallas_call(
        body,
        grid=(n // BLK,),
        in_specs=[pl.BlockSpec((BLK, kdim), lambda i: (i, 0)),
                  pl.BlockSpec((kdim, c), lambda i: (0, 0)),
                  pl.BlockSpec((1, c), lambda i: (0, 0))],
        out_specs=pl.BlockSpec((BLK, c), lambda i: (i, 0)),
        out_shape=jax.ShapeDtypeStruct((n, c), jnp.float32),
    )(x, w, b)


def _tc_comb(p0, p1, d0, d1):
    n, c = p0.shape

    def body(p0r, p1r, d0r, d1r, o_ref):
        d = d0r[...] + d1r[...]
        num = p0r[...] + p1r[...]
        pos = d > 0
        o_ref[...] = jnp.where(pos, num / jnp.where(pos, d, 1.0), 0.0)

    return pl.pallas_call(
        body,
        grid=(n // BLK,),
        in_specs=[pl.BlockSpec((BLK, c), lambda i: (i, 0)),
                  pl.BlockSpec((BLK, c), lambda i: (i, 0)),
                  pl.BlockSpec((BLK, 1), lambda i: (i, 0)),
                  pl.BlockSpec((BLK, 1), lambda i: (i, 0))],
        out_specs=pl.BlockSpec((BLK, c), lambda i: (i, 0)),
        out_shape=jax.ShapeDtypeStruct((n, c), jnp.float32),
    )(p0, p1, d0, d1)


def _tc_comb_relu(q0, q1, d0, d1):
    n, c = q0.shape

    def body(q0r, q1r, d0r, d1r, o_ref):
        d = d0r[...] + d1r[...]
        num = q0r[...] + q1r[...]
        pos = d > 0
        xv = jnp.where(pos, num / jnp.where(pos, d, 1.0), 0.0)
        o_ref[...] = jnp.maximum(xv, 0.0)

    return pl.pallas_call(
        body,
        grid=(n // BLK,),
        in_specs=[pl.BlockSpec((BLK, c), lambda i: (i, 0)),
                  pl.BlockSpec((BLK, c), lambda i: (i, 0)),
                  pl.BlockSpec((BLK, 1), lambda i: (i, 0)),
                  pl.BlockSpec((BLK, 1), lambda i: (i, 0))],
        out_specs=pl.BlockSpec((BLK, c), lambda i: (i, 0)),
        out_shape=jax.ShapeDtypeStruct((n, c), jnp.float32),
    )(q0, q1, d0, d1)


def _tc_comb_mm_mask(p0, p1, d0, d1, w, b):
    n, c = p0.shape
    c2 = w.shape[1]

    def body(p0r, p1r, d0r, d1r, wr, br, mo, yo):
        d = d0r[...] + d1r[...]
        num = p0r[...] + p1r[...]
        pos = d > 0
        m = jnp.where(pos, num / jnp.where(pos, d, 1.0), 0.0)
        mo[...] = m
        y = (jnp.dot(m, wr[...],
                     preferred_element_type=jnp.float32) + br[...])
        yo[...] = jnp.where(pos, y, 0.0)

    return pl.pallas_call(
        body,
        grid=(n // BLK,),
        in_specs=[pl.BlockSpec((BLK, c), lambda i: (i, 0)),
                  pl.BlockSpec((BLK, c), lambda i: (i, 0)),
                  pl.BlockSpec((BLK, 1), lambda i: (i, 0)),
                  pl.BlockSpec((BLK, 1), lambda i: (i, 0)),
                  pl.BlockSpec((c, c2), lambda i: (0, 0)),
                  pl.BlockSpec((1, c2), lambda i: (0, 0))],
        out_specs=[pl.BlockSpec((BLK, c), lambda i: (i, 0)),
                   pl.BlockSpec((BLK, c2), lambda i: (i, 0))],
        out_shape=[jax.ShapeDtypeStruct((n, c), jnp.float32),
                   jax.ShapeDtypeStruct((n, c2), jnp.float32)],
    )(p0, p1, d0, d1, w, b)


def kernel(X, hg, v2e_weight, e2v_weight, W1, b1, W2, b2):
    vid = hg[0]
    eid = hg[1]
    padn = NNZP - NNZ
    pad_idx = jnp.full((padn,), NP - 1, jnp.int32)
    pad_w = jnp.zeros((padn,), jnp.float32)
    shape4 = (2, 16, NCHUNK, CHUNK)
    vid_p = jnp.concatenate([vid, pad_idx]).reshape(shape4)
    eid_p = jnp.concatenate([eid, pad_idx]).reshape(shape4)
    wv_p = jnp.concatenate([v2e_weight, pad_w]).reshape(shape4)
    we_p = jnp.concatenate([e2v_weight, pad_w]).reshape(shape4)
    X_p = jnp.pad(X, ((0, NP - NV), (0, 0)))
    b1r = b1.reshape(1, -1)
    b2r = b2.reshape(1, -1)

    de0, de1, dv0, dv1 = _degree_kernel()(vid_p, eid_p, wv_p, we_p)
    de0c, de1c = de0[:, :1], de1[:, :1]
    dv0c, dv1c = dv0[:, :1], dv1[:, :1]

    t1 = _tc_mm(X_p, W1, b1r)
    p0, p1 = _accum_kernel(64)(t1, vid_p, eid_p, wv_p)
    xe1 = _tc_comb(p0, p1, de0c, de1c)
    q0, q1 = _accum_kernel(64)(xe1, eid_p, vid_p, we_p)
    xn1 = _tc_comb_relu(q0, q1, dv0c, dv1c)
    r0, r1 = _accum_kernel(64)(xn1, vid_p, eid_p, wv_p)
    m1, xe2 = _tc_comb_mm_mask(r0, r1, de0c, de1c, W2, b2r)
    s0, s1 = _accum_kernel(64)(m1, eid_p, vid_p, we_p)
    _, xn2 = _tc_comb_mm_mask(s0, s1, dv0c, dv1c, W2, b2r)

    return (xn1[:NV], xe1[:NV], xn2[:NV], xe2[:NV])


# speedup vs baseline: 5.2473x; 1.5369x over previous
import functools

import jax
import jax.numpy as jnp
from jax import lax
from jax.experimental import pallas as pl
from jax.experimental.pallas import tpu as pltpu
from jax.experimental.pallas import tpu_sc as plsc

NV = 10000
NP = 10240
NNZ = 320000
CHUNK = 128
NCHUNK = 80
PER_TILE = CHUNK * NCHUNK
NNZP = PER_TILE * 32
RPT = NP // 16
BLK = 256


def _sc_mesh():
    return plsc.VectorSubcoreMesh(core_axis_name="c", subcore_axis_name="s")


@functools.cache
def _degree_kernel():
    out = [jax.ShapeDtypeStruct((NP, 16), jnp.float32) for _ in range(4)]
    scratch = [
        pltpu.VMEM((NCHUNK, CHUNK), jnp.int32),
        pltpu.VMEM((NCHUNK, CHUNK), jnp.int32),
        pltpu.VMEM((NCHUNK, CHUNK), jnp.float32),
        pltpu.VMEM((NCHUNK, CHUNK), jnp.float32),
        pltpu.VMEM((CHUNK, 16), jnp.float32),
        pltpu.VMEM((CHUNK, 16), jnp.float32),
        pltpu.VMEM_SHARED((NP, 16), jnp.float32),
        pltpu.VMEM_SHARED((NP, 16), jnp.float32),
    ]

    @functools.partial(
        pl.kernel, out_type=out, mesh=_sc_mesh(), scratch_types=scratch,
        compiler_params=pltpu.CompilerParams(use_tc_tiling_on_sc=False))
    def k(vid_h, eid_h, wv_h, we_h, de0, de1, dv0, dv1,
          eidb, vidb, wvb, web, mbe, mbv, acc_de, acc_dv):
        cid = lax.axis_index("c")
        sid = lax.axis_index("s")
        pltpu.sync_copy(eid_h.at[cid, sid], eidb)
        pltpu.sync_copy(vid_h.at[cid, sid], vidb)
        pltpu.sync_copy(wv_h.at[cid, sid], wvb)
        pltpu.sync_copy(we_h.at[cid, sid], web)

        zero = jnp.zeros((16,), jnp.float32)

        @pl.loop(0, CHUNK)
        def _(r):
            mbe[r, :] = zero
        base = sid * RPT
        for kk in range(RPT // CHUNK):
            pltpu.sync_copy(mbe, acc_de.at[pl.ds(base + kk * CHUNK, CHUNK)])
            pltpu.sync_copy(mbe, acc_dv.at[pl.ds(base + kk * CHUNK, CHUNK)])
        plsc.subcore_barrier()

        @pl.loop(0, NCHUNK)
        def _(g):
            @pl.loop(0, CHUNK // 16)
            def _(kk):
                wv_vec = wvb[g, pl.ds(kk * 16, 16)]
                we_vec = web[g, pl.ds(kk * 16, 16)]
                for i in range(16):
                    mbe[kk * 16 + i, :] = jnp.full((16,), wv_vec[i],
                                                   jnp.float32)
                    mbv[kk * 16 + i, :] = jnp.full((16,), we_vec[i],
                                                   jnp.float32)
            pltpu.sync_copy(mbe, acc_de.at[eidb.at[g]], add=True)
            pltpu.sync_copy(mbv, acc_dv.at[vidb.at[g]], add=True)

        plsc.subcore_barrier()
        for kk in range(RPT // CHUNK):
            sl = pl.ds(base + kk * CHUNK, CHUNK)
            pltpu.sync_copy(acc_de.at[sl], mbe)
            pltpu.sync_copy(acc_dv.at[sl], mbv)

            @pl.when(cid == 0)
            def _():
                pltpu.sync_copy(mbe, de0.at[sl])
                pltpu.sync_copy(mbv, dv0.at[sl])

            @pl.when(cid == 1)
            def _():
                pltpu.sync_copy(mbe, de1.at[sl])
                pltpu.sync_copy(mbv, dv1.at[sl])

    return k


@functools.cache
def _accum_kernel(C):
    out = [jax.ShapeDtypeStruct((NP, C), jnp.float32) for _ in range(2)]
    scratch = [
        pltpu.VMEM((NCHUNK, CHUNK), jnp.int32),
        pltpu.VMEM((NCHUNK, CHUNK), jnp.int32),
        pltpu.VMEM((NCHUNK, CHUNK), jnp.float32),
        pltpu.VMEM((CHUNK, C), jnp.float32),
        pltpu.VMEM((CHUNK, C), jnp.float32),
        pltpu.VMEM_SHARED((NP, C), jnp.float32),
        pltpu.SemaphoreType.DMA,
        pltpu.SemaphoreType.DMA,
    ]

    @functools.partial(
        pl.kernel, out_type=out, mesh=_sc_mesh(), scratch_types=scratch,
        compiler_params=pltpu.CompilerParams(use_tc_tiling_on_sc=False))
    def k(table_h, src_h, dst_h, w_h, out0, out1,
          srcb, dstb, wb, gbuf0, gbuf1, acc, sem0, sem1):
        cid = lax.axis_index("c")
        sid = lax.axis_index("s")
        pltpu.sync_copy(src_h.at[cid, sid], srcb)
        pltpu.sync_copy(dst_h.at[cid, sid], dstb)
        pltpu.sync_copy(w_h.at[cid, sid], wb)

        zero = jnp.zeros((16,), jnp.float32)

        @pl.loop(0, CHUNK)
        def _(r):
            for j in range(C // 16):
                gbuf0[r, pl.ds(16 * j, 16)] = zero
        base = sid * RPT
        for kk in range(RPT // CHUNK):
            pltpu.sync_copy(gbuf0, acc.at[pl.ds(base + kk * CHUNK, CHUNK)])
        plsc.subcore_barrier()

        def fire(g, buf, sem):
            pltpu.async_copy(table_h.at[srcb.at[g]], buf, sem)

        def drain(buf, sem):
            pltpu.make_async_copy(table_h.at[srcb.at[0]], buf, sem).wait()

        def process(g, buf):
            @pl.loop(0, CHUNK // 16)
            def _(kk):
                w_vec = wb[g, pl.ds(kk * 16, 16)]
                for i in range(16):
                    s = jnp.full((16,), w_vec[i], jnp.float32)
                    r = kk * 16 + i
                    for j in range(C // 16):
                        buf[r, pl.ds(16 * j, 16)] = (
                            buf[r, pl.ds(16 * j, 16)] * s)

            pltpu.sync_copy(buf, acc.at[dstb.at[g]], add=True)

        fire(0, gbuf0, sem0)

        @pl.loop(0, NCHUNK // 2)
        def _(h):
            g = h * 2
            fire(g + 1, gbuf1, sem1)
            drain(gbuf0, sem0)
            process(g, gbuf0)

            @pl.when(g + 2 < NCHUNK)
            def _():
                fire(g + 2, gbuf0, sem0)

            drain(gbuf1, sem1)
            process(g + 1, gbuf1)

        plsc.subcore_barrier()
        for kk in range(RPT // CHUNK):
            sl = pl.ds(base + kk * CHUNK, CHUNK)
            pltpu.sync_copy(acc.at[sl], gbuf0)

            @pl.when(cid == 0)
            def _():
                pltpu.sync_copy(gbuf0, out0.at[sl])

            @pl.when(cid == 1)
            def _():
                pltpu.sync_copy(gbuf0, out1.at[sl])

    return k


def _tc_mm(x, w, b):
    n, kdim = x.shape
    c = w.shape[1]

    def body(x_ref, w_ref, b_ref, o_ref):
        o_ref[...] = (jnp.dot(x_ref[...], w_ref[...],
                              preferred_element_type=jnp.float32) + b_ref[...])

    return pl.pallas_call(
        body,
        grid=(n // BLK,),
        in_specs=[pl.BlockSpec((BLK, kdim), lambda i: (i, 0)),
                  pl.BlockSpec((kdim, c), lambda i: (0, 0)),
                  pl.BlockSpec((1, c), lambda i: (0, 0))],
        out_specs=pl.BlockSpec((BLK, c), lambda i: (i, 0)),
        out_shape=jax.ShapeDtypeStruct((n, c), jnp.float32),
    )(x, w, b)


def _tc_comb(p0, p1, d0, d1):
    n, c = p0.shape

    def body(p0r, p1r, d0r, d1r, o_ref):
        d = d0r[...] + d1r[...]
        num = p0r[...] + p1r[...]
        pos = d > 0
        o_ref[...] = jnp.where(pos, num / jnp.where(pos, d, 1.0), 0.0)

    return pl.pallas_call(
        body,
        grid=(n // BLK,),
        in_specs=[pl.BlockSpec((BLK, c), lambda i: (i, 0)),
                  pl.BlockSpec((BLK, c), lambda i: (i, 0)),
                  pl.BlockSpec((BLK, 1), lambda i: (i, 0)),
                  pl.BlockSpec((BLK, 1), lambda i: (i, 0))],
        out_specs=pl.BlockSpec((BLK, c), lambda i: (i, 0)),
        out_shape=jax.ShapeDtypeStruct((n, c), jnp.float32),
    )(p0, p1, d0, d1)


def _tc_comb_relu(q0, q1, d0, d1):
    n, c = q0.shape

    def body(q0r, q1r, d0r, d1r, o_ref):
        d = d0r[...] + d1r[...]
        num = q0r[...] + q1r[...]
        pos = d > 0
        xv = jnp.where(pos, num / jnp.where(pos, d, 1.0), 0.0)
        o_ref[...] = jnp.maximum(xv, 0.0)

    return pl.pallas_call(
        body,
        grid=(n // BLK,),
        in_specs=[pl.BlockSpec((BLK, c), lambda i: (i, 0)),
                  pl.BlockSpec((BLK, c), lambda i: (i, 0)),
                  pl.BlockSpec((BLK, 1), lambda i: (i, 0)),
                  pl.BlockSpec((BLK, 1), lambda i: (i, 0))],
        out_specs=pl.BlockSpec((BLK, c), lambda i: (i, 0)),
        out_shape=jax.ShapeDtypeStruct((n, c), jnp.float32),
    )(q0, q1, d0, d1)


def _tc_comb_mm_mask(p0, p1, d0, d1, w, b):
    n, c = p0.shape
    c2 = w.shape[1]

    def body(p0r, p1r, d0r, d1r, wr, br, mo, yo):
        d = d0r[...] + d1r[...]
        num = p0r[...] + p1r[...]
        pos = d > 0
        m = jnp.where(pos, num / jnp.where(pos, d, 1.0), 0.0)
        mo[...] = m
        y = (jnp.dot(m, wr[...],
                     preferred_element_type=jnp.float32) + br[...])
        yo[...] = jnp.where(pos, y, 0.0)

    return pl.pallas_call(
        body,
        grid=(n // BLK,),
        in_specs=[pl.BlockSpec((BLK, c), lambda i: (i, 0)),
                  pl.BlockSpec((BLK, c), lambda i: (i, 0)),
                  pl.BlockSpec((BLK, 1), lambda i: (i, 0)),
                  pl.BlockSpec((BLK, 1), lambda i: (i, 0)),
                  pl.BlockSpec((c, c2), lambda i: (0, 0)),
                  pl.BlockSpec((1, c2), lambda i: (0, 0))],
        out_specs=[pl.BlockSpec((BLK, c), lambda i: (i, 0)),
                   pl.BlockSpec((BLK, c2), lambda i: (i, 0))],
        out_shape=[jax.ShapeDtypeStruct((n, c), jnp.float32),
                   jax.ShapeDtypeStruct((n, c2), jnp.float32)],
    )(p0, p1, d0, d1, w, b)


def kernel(X, hg, v2e_weight, e2v_weight, W1, b1, W2, b2):
    vid = hg[0]
    eid = hg[1]
    padn = NNZP - NNZ
    pad_idx = jnp.full((padn,), NP - 1, jnp.int32)
    pad_w = jnp.zeros((padn,), jnp.float32)
    shape4 = (2, 16, NCHUNK, CHUNK)
    vid_p = jnp.concatenate([vid, pad_idx]).reshape(shape4)
    eid_p = jnp.concatenate([eid, pad_idx]).reshape(shape4)
    wv_p = jnp.concatenate([v2e_weight, pad_w]).reshape(shape4)
    we_p = jnp.concatenate([e2v_weight, pad_w]).reshape(shape4)
    X_p = jnp.pad(X, ((0, NP - NV), (0, 0)))
    b1r = b1.reshape(1, -1)
    b2r = b2.reshape(1, -1)

    de0, de1, dv0, dv1 = _degree_kernel()(vid_p, eid_p, wv_p, we_p)
    de0c, de1c = de0[:, :1], de1[:, :1]
    dv0c, dv1c = dv0[:, :1], dv1[:, :1]

    t1 = _tc_mm(X_p, W1, b1r)
    p0, p1 = _accum_kernel(64)(t1, vid_p, eid_p, wv_p)
    xe1 = _tc_comb(p0, p1, de0c, de1c)
    q0, q1 = _accum_kernel(64)(xe1, eid_p, vid_p, we_p)
    xn1 = _tc_comb_relu(q0, q1, dv0c, dv1c)
    r0, r1 = _accum_kernel(64)(xn1, vid_p, eid_p, wv_p)
    m1, xe2 = _tc_comb_mm_mask(r0, r1, de0c, de1c, W2, b2r)
    s0, s1 = _accum_kernel(64)(m1, eid_p, vid_p, we_p)
    _, xn2 = _tc_comb_mm_mask(s0, s1, dv0c, dv1c, W2, b2r)

    return (xn1[:NV], xe1[:NV], xn2[:NV], xe2[:NV])


# speedup vs baseline: 5.4522x; 1.0390x over previous
import functools

import jax
import jax.numpy as jnp
from jax import lax
from jax.experimental import pallas as pl
from jax.experimental.pallas import tpu as pltpu
from jax.experimental.pallas import tpu_sc as plsc

NV = 10000
NP = 10240
NNZ = 320000
CHUNK = 128
NCHUNK = 80
PER_TILE = CHUNK * NCHUNK
NNZP = PER_TILE * 32
RPT = NP // 16
BLK = 256


def _sc_mesh():
    return plsc.VectorSubcoreMesh(core_axis_name="c", subcore_axis_name="s")


@functools.cache
def _degree_kernel():
    out = [jax.ShapeDtypeStruct((NP, 16), jnp.float32) for _ in range(4)]
    scratch = [
        pltpu.VMEM((NCHUNK, CHUNK), jnp.int32),
        pltpu.VMEM((NCHUNK, CHUNK), jnp.int32),
        pltpu.VMEM((NCHUNK, CHUNK), jnp.float32),
        pltpu.VMEM((NCHUNK, CHUNK), jnp.float32),
        pltpu.VMEM((CHUNK, 16), jnp.float32),
        pltpu.VMEM((CHUNK, 16), jnp.float32),
        pltpu.VMEM_SHARED((NP, 16), jnp.float32),
        pltpu.VMEM_SHARED((NP, 16), jnp.float32),
    ]

    @functools.partial(
        pl.kernel, out_type=out, mesh=_sc_mesh(), scratch_types=scratch,
        compiler_params=pltpu.CompilerParams(use_tc_tiling_on_sc=False))
    def k(vid_h, eid_h, wv_h, we_h, de0, de1, dv0, dv1,
          eidb, vidb, wvb, web, mbe, mbv, acc_de, acc_dv):
        cid = lax.axis_index("c")
        sid = lax.axis_index("s")
        pltpu.sync_copy(eid_h.at[cid, sid], eidb)
        pltpu.sync_copy(vid_h.at[cid, sid], vidb)
        pltpu.sync_copy(wv_h.at[cid, sid], wvb)
        pltpu.sync_copy(we_h.at[cid, sid], web)

        zero = jnp.zeros((16,), jnp.float32)

        @pl.loop(0, CHUNK)
        def _(r):
            mbe[r, :] = zero
        base = sid * RPT
        for kk in range(RPT // CHUNK):
            pltpu.sync_copy(mbe, acc_de.at[pl.ds(base + kk * CHUNK, CHUNK)])
            pltpu.sync_copy(mbe, acc_dv.at[pl.ds(base + kk * CHUNK, CHUNK)])
        plsc.subcore_barrier()

        @pl.loop(0, NCHUNK)
        def _(g):
            @pl.loop(0, CHUNK // 16)
            def _(kk):
                wv_vec = wvb[g, pl.ds(kk * 16, 16)]
                we_vec = web[g, pl.ds(kk * 16, 16)]
                for i in range(16):
                    mbe[kk * 16 + i, :] = jnp.full((16,), wv_vec[i],
                                                   jnp.float32)
                    mbv[kk * 16 + i, :] = jnp.full((16,), we_vec[i],
                                                   jnp.float32)
            pltpu.sync_copy(mbe, acc_de.at[eidb.at[g]], add=True)
            pltpu.sync_copy(mbv, acc_dv.at[vidb.at[g]], add=True)

        plsc.subcore_barrier()
        for kk in range(RPT // CHUNK):
            sl = pl.ds(base + kk * CHUNK, CHUNK)
            pltpu.sync_copy(acc_de.at[sl], mbe)
            pltpu.sync_copy(acc_dv.at[sl], mbv)

            @pl.when(cid == 0)
            def _():
                pltpu.sync_copy(mbe, de0.at[sl])
                pltpu.sync_copy(mbv, dv0.at[sl])

            @pl.when(cid == 1)
            def _():
                pltpu.sync_copy(mbe, de1.at[sl])
                pltpu.sync_copy(mbv, dv1.at[sl])

    return k


@functools.cache
def _accum_kernel(C):
    out = [jax.ShapeDtypeStruct((NP, C), jnp.float32) for _ in range(2)]
    scratch = [
        pltpu.VMEM((NCHUNK, CHUNK), jnp.int32),
        pltpu.VMEM((NCHUNK, CHUNK), jnp.int32),
        pltpu.VMEM((NCHUNK, CHUNK), jnp.float32),
        pltpu.VMEM((CHUNK, C), jnp.float32),
        pltpu.VMEM((CHUNK, C), jnp.float32),
        pltpu.VMEM((CHUNK, C), jnp.float32),
        pltpu.VMEM((CHUNK, C), jnp.float32),
        pltpu.VMEM_SHARED((NP, C), jnp.float32),
    ] + [pltpu.SemaphoreType.DMA] * 8

    @functools.partial(
        pl.kernel, out_type=out, mesh=_sc_mesh(), scratch_types=scratch,
        compiler_params=pltpu.CompilerParams(use_tc_tiling_on_sc=False))
    def k(table_h, src_h, dst_h, w_h, out0, out1,
          srcb, dstb, wb, b0, b1, b2, b3, acc,
          gs0, gs1, gs2, gs3, ss0, ss1, ss2, ss3):
        bufs = (b0, b1, b2, b3)
        gs = (gs0, gs1, gs2, gs3)
        ss = (ss0, ss1, ss2, ss3)
        cid = lax.axis_index("c")
        sid = lax.axis_index("s")
        pltpu.sync_copy(src_h.at[cid, sid], srcb)
        pltpu.sync_copy(dst_h.at[cid, sid], dstb)
        pltpu.sync_copy(w_h.at[cid, sid], wb)

        zero = jnp.zeros((16,), jnp.float32)

        @pl.loop(0, CHUNK)
        def _(r):
            for j in range(C // 16):
                b0[r, pl.ds(16 * j, 16)] = zero
        base = sid * RPT
        for kk in range(RPT // CHUNK):
            pltpu.sync_copy(b0, acc.at[pl.ds(base + kk * CHUNK, CHUNK)])
        plsc.subcore_barrier()

        def fire(g, buf, sem):
            pltpu.async_copy(table_h.at[srcb.at[g]], buf, sem)

        def drain_g(buf, sem):
            pltpu.make_async_copy(table_h.at[srcb.at[0]], buf, sem).wait()

        def drain_s(buf, sem):
            pltpu.make_async_copy(table_h.at[srcb.at[0]], buf, sem).wait()

        def scale(g, buf):
            @pl.loop(0, CHUNK // 16)
            def _(kk):
                w_vec = wb[g, pl.ds(kk * 16, 16)]
                for i in range(16):
                    s = jnp.full((16,), w_vec[i], jnp.float32)
                    r = kk * 16 + i
                    for j in range(C // 16):
                        buf[r, pl.ds(16 * j, 16)] = (
                            buf[r, pl.ds(16 * j, 16)] * s)

        fire(0, bufs[0], gs[0])
        fire(1, bufs[1], gs[1])

        @pl.loop(0, NCHUNK // 4)
        def _(h):
            for b in range(4):
                g = h * 4 + b
                bb = (b + 2) % 4

                @pl.when(g >= 2)
                def _():
                    drain_s(bufs[bb], ss[bb])

                @pl.when(g + 2 < NCHUNK)
                def _():
                    fire(g + 2, bufs[bb], gs[bb])

                drain_g(bufs[b], gs[b])
                scale(g, bufs[b])
                pltpu.async_copy(bufs[b], acc.at[dstb.at[g]], ss[b],
                                 add=True)

        drain_s(bufs[(NCHUNK - 2) % 4], ss[(NCHUNK - 2) % 4])
        drain_s(bufs[(NCHUNK - 1) % 4], ss[(NCHUNK - 1) % 4])
        plsc.subcore_barrier()
        for kk in range(RPT // CHUNK):
            sl = pl.ds(base + kk * CHUNK, CHUNK)
            pltpu.sync_copy(acc.at[sl], b0)

            @pl.when(cid == 0)
            def _():
                pltpu.sync_copy(b0, out0.at[sl])

            @pl.when(cid == 1)
            def _():
                pltpu.sync_copy(b0, out1.at[sl])

    return k


def _tc_mm(x, w, b):
    n, kdim = x.shape
    c = w.shape[1]

    def body(x_ref, w_ref, b_ref, o_ref):
        o_ref[...] = (jnp.dot(x_ref[...], w_ref[...],
                              preferred_element_type=jnp.float32) + b_ref[...])

    return pl.pallas_call(
        body,
        grid=(n // BLK,),
        in_specs=[pl.BlockSpec((BLK, kdim), lambda i: (i, 0)),
                  pl.BlockSpec((kdim, c), lambda i: (0, 0)),
                  pl.BlockSpec((1, c), lambda i: (0, 0))],
        out_specs=pl.BlockSpec((BLK, c), lambda i: (i, 0)),
        out_shape=jax.ShapeDtypeStruct((n, c), jnp.float32),
    )(x, w, b)


def _tc_comb(p0, p1, d0, d1):
    n, c = p0.shape

    def body(p0r, p1r, d0r, d1r, o_ref):
        d = d0r[...] + d1r[...]
        num = p0r[...] + p1r[...]
        pos = d > 0
        o_ref[...] = jnp.where(pos, num / jnp.where(pos, d, 1.0), 0.0)

    return pl.pallas_call(
        body,
        grid=(n // BLK,),
        in_specs=[pl.BlockSpec((BLK, c), lambda i: (i, 0)),
                  pl.BlockSpec((BLK, c), lambda i: (i, 0)),
                  pl.BlockSpec((BLK, 1), lambda i: (i, 0)),
                  pl.BlockSpec((BLK, 1), lambda i: (i, 0))],
        out_specs=pl.BlockSpec((BLK, c), lambda i: (i, 0)),
        out_shape=jax.ShapeDtypeStruct((n, c), jnp.float32),
    )(p0, p1, d0, d1)


def _tc_comb_relu(q0, q1, d0, d1):
    n, c = q0.shape

    def body(q0r, q1r, d0r, d1r, o_ref):
        d = d0r[...] + d1r[...]
        num = q0r[...] + q1r[...]
        pos = d > 0
        xv = jnp.where(pos, num / jnp.where(pos, d, 1.0), 0.0)
        o_ref[...] = jnp.maximum(xv, 0.0)

    return pl.pallas_call(
        body,
        grid=(n // BLK,),
        in_specs=[pl.BlockSpec((BLK, c), lambda i: (i, 0)),
                  pl.BlockSpec((BLK, c), lambda i: (i, 0)),
                  pl.BlockSpec((BLK, 1), lambda i: (i, 0)),
                  pl.BlockSpec((BLK, 1), lambda i: (i, 0))],
        out_specs=pl.BlockSpec((BLK, c), lambda i: (i, 0)),
        out_shape=jax.ShapeDtypeStruct((n, c), jnp.float32),
    )(q0, q1, d0, d1)


def _tc_comb_mm_mask(p0, p1, d0, d1, w, b):
    n, c = p0.shape
    c2 = w.shape[1]

    def body(p0r, p1r, d0r, d1r, wr, br, mo, yo):
        d = d0r[...] + d1r[...]
        num = p0r[...] + p1r[...]
        pos = d > 0
        m = jnp.where(pos, num / jnp.where(pos, d, 1.0), 0.0)
        mo[...] = m
        y = (jnp.dot(m, wr[...],
                     preferred_element_type=jnp.float32) + br[...])
        yo[...] = jnp.where(pos, y, 0.0)

    return pl.pallas_call(
        body,
        grid=(n // BLK,),
        in_specs=[pl.BlockSpec((BLK, c), lambda i: (i, 0)),
                  pl.BlockSpec((BLK, c), lambda i: (i, 0)),
                  pl.BlockSpec((BLK, 1), lambda i: (i, 0)),
                  pl.BlockSpec((BLK, 1), lambda i: (i, 0)),
                  pl.BlockSpec((c, c2), lambda i: (0, 0)),
                  pl.BlockSpec((1, c2), lambda i: (0, 0))],
        out_specs=[pl.BlockSpec((BLK, c), lambda i: (i, 0)),
                   pl.BlockSpec((BLK, c2), lambda i: (i, 0))],
        out_shape=[jax.ShapeDtypeStruct((n, c), jnp.float32),
                   jax.ShapeDtypeStruct((n, c2), jnp.float32)],
    )(p0, p1, d0, d1, w, b)


def kernel(X, hg, v2e_weight, e2v_weight, W1, b1, W2, b2):
    vid = hg[0]
    eid = hg[1]
    padn = NNZP - NNZ
    pad_idx = jnp.full((padn,), NP - 1, jnp.int32)
    pad_w = jnp.zeros((padn,), jnp.float32)
    shape4 = (2, 16, NCHUNK, CHUNK)
    vid_p = jnp.concatenate([vid, pad_idx]).reshape(shape4)
    eid_p = jnp.concatenate([eid, pad_idx]).reshape(shape4)
    wv_p = jnp.concatenate([v2e_weight, pad_w]).reshape(shape4)
    we_p = jnp.concatenate([e2v_weight, pad_w]).reshape(shape4)
    X_p = jnp.pad(X, ((0, NP - NV), (0, 0)))
    b1r = b1.reshape(1, -1)
    b2r = b2.reshape(1, -1)

    de0, de1, dv0, dv1 = _degree_kernel()(vid_p, eid_p, wv_p, we_p)
    de0c, de1c = de0[:, :1], de1[:, :1]
    dv0c, dv1c = dv0[:, :1], dv1[:, :1]

    t1 = _tc_mm(X_p, W1, b1r)
    p0, p1 = _accum_kernel(64)(t1, vid_p, eid_p, wv_p)
    xe1 = _tc_comb(p0, p1, de0c, de1c)
    q0, q1 = _accum_kernel(64)(xe1, eid_p, vid_p, we_p)
    xn1 = _tc_comb_relu(q0, q1, dv0c, dv1c)
    r0, r1 = _accum_kernel(64)(xn1, vid_p, eid_p, wv_p)
    m1, xe2 = _tc_comb_mm_mask(r0, r1, de0c, de1c, W2, b2r)
    s0, s1 = _accum_kernel(64)(m1, eid_p, vid_p, we_p)
    _, xn2 = _tc_comb_mm_mask(s0, s1, dv0c, dv1c, W2, b2r)

    return (xn1[:NV], xe1[:NV], xn2[:NV], xe2[:NV])
